# baseline (speedup 1.0000x reference)
import functools

import jax
import jax.numpy as jnp
from jax import lax
from jax.experimental import pallas as pl
from jax.experimental.pallas import tpu as pltpu
from jax.experimental.pallas import tpu_sc as plsc

_NUM_CORES = 2
_NUM_SUBCORES = 16
_NW = _NUM_CORES * _NUM_SUBCORES
_ICH = 128


@functools.lru_cache(maxsize=None)
def _make_sc_gather(B, D, N):
    bpw = B // _NW
    nch = bpw // _ICH
    mesh = plsc.VectorSubcoreMesh(core_axis_name="c", subcore_axis_name="s")

    @functools.partial(
        pl.kernel,
        mesh=mesh,
        out_type=[jax.ShapeDtypeStruct((B * D,), jnp.float32)] * 2,
        scratch_types=[
            pltpu.VMEM((nch, _ICH), jnp.int32),
            pltpu.VMEM((nch, _ICH), jnp.int32),
            pltpu.VMEM((nch, _ICH), jnp.int32),
            pltpu.VMEM((D * bpw,), jnp.float32),
            [pltpu.VMEM((_ICH, 16), jnp.float32) for _ in range(nch)],
            pltpu.SemaphoreType.DMA,
        ],
        compiler_params=pltpu.CompilerParams(
            use_tc_tiling_on_sc=False, needs_layout_passes=False),
    )
    def sc_gather(xg_hbm, hidx_hbm, tidx_hbm,
                  ht_out, tt_out,
                  hiv, tiv, civ, stage, gbufs, sem):
        wid = lax.axis_index("s") * _NUM_CORES + lax.axis_index("c")
        cbase = wid * nch
        pltpu.sync_copy(hidx_hbm.at[pl.ds(cbase, nch)], hiv)
        pltpu.sync_copy(tidx_hbm.at[pl.ds(cbase, nch)], tiv)
        lanes = lax.broadcasted_iota(jnp.int32, (16,), 0)
        n16 = N // 16

        for iv, out in ((hiv, ht_out), (tiv, tt_out)):
            def per_dim(d, _, iv=iv):
                dn16 = d * n16
                for c in range(nch):
                    for q in range(_ICH // 16):
                        sl = pl.ds(q * 16, 16)
                        civ[c, sl] = (iv[c, sl] >> 4) + dn16
                cps = []
                for c in range(nch):
                    cps.append(pltpu.async_copy(
                        xg_hbm.at[civ.at[c]], gbufs[c], sem))
                for cp in cps:
                    cp.wait()
                for c in range(nch):
                    for q in range(_ICH // 16):
                        sl = pl.ds(q * 16, 16)
                        jv = lanes + (q * 16)
                        mv = iv[c, sl] & 15
                        vals = plsc.load_gather(gbufs[c], [jv, mv])
                        stage[pl.ds(d * bpw + c * _ICH + q * 16, 16)] = vals
                return 0

            lax.fori_loop(0, D, per_dim, 0)
            pltpu.sync_copy(stage, out.at[pl.ds(wid * D * bpw, D * bpw)])

    return sc_gather


def _tc_score_body(ht_ref, tt_ref, ri_ref, relt_ref, w_ref, cb_ref, gt_ref,
                   s_ref, l2_ref, *, out_ch, n_rel, l2_scale):
    i = pl.program_id(0)
    h = ht_ref[0]
    t = tt_ref[0]
    cols = ht_ref.shape[2]
    onehot = (lax.broadcasted_iota(jnp.int32, (n_rel, cols), 0).astype(
        jnp.float32) == ri_ref[...]).astype(jnp.float32)
    r = jnp.dot(relt_ref[...], onehot, preferred_element_type=jnp.float32)
    acc = None
    for o in range(out_ch):
        pre = h * w_ref[o, 0] + r * w_ref[o, 1] + t * w_ref[o, 2] + cb_ref[o]
        z = jnp.maximum(pre, 0.0)
        term = z * gt_ref[:, pl.ds(o, 1)]
        acc = term if acc is None else acc + term
    s_ref[...] = jnp.sum(acc, axis=0, keepdims=True)
    part = (jnp.sum(h * h) + jnp.sum(t * t) + jnp.sum(r * r)) * l2_scale

    @pl.when(i == 0)
    def _():
        l2_ref[0, 0] = part

    @pl.when(i > 0)
    def _():
        l2_ref[0, 0] = l2_ref[0, 0] + part


def _tc_score(ht, tt, ri, relt, w, cb, gt, *, out_ch, interpret=False):
    NW, D, tile = ht.shape
    B = NW * tile
    n_rel = relt.shape[1]
    body = functools.partial(
        _tc_score_body, out_ch=out_ch, n_rel=n_rel,
        l2_scale=1.0 / (3.0 * B * D))
    return pl.pallas_call(
        body,
        grid=(NW,),
        in_specs=[
            pl.BlockSpec((1, D, tile), lambda i: (i, 0, 0)),
            pl.BlockSpec((1, D, tile), lambda i: (i, 0, 0)),
            pl.BlockSpec((1, tile), lambda i: (0, i)),
            pl.BlockSpec((D, n_rel), lambda i: (0, 0)),
            pl.BlockSpec(memory_space=pltpu.SMEM),
            pl.BlockSpec(memory_space=pltpu.SMEM),
            pl.BlockSpec((D, out_ch), lambda i: (0, 0)),
        ],
        out_specs=[
            pl.BlockSpec((1, tile), lambda i: (0, i)),
            pl.BlockSpec(memory_space=pltpu.SMEM),
        ],
        out_shape=[
            jax.ShapeDtypeStruct((1, B), jnp.float32),
            jax.ShapeDtypeStruct((1, 1), jnp.float32),
        ],
        compiler_params=pltpu.CompilerParams(
            dimension_semantics=("arbitrary",)),
        interpret=interpret,
    )(ht, tt, ri, relt, w, cb, gt)


def kernel(x, rel_emb, head_index, rel_type, tail_index, conv_w, conv_b, fc_w):
    B = head_index.shape[0]
    N, D = x.shape
    out_ch = conv_w.shape[0]

    hi = head_index.astype(jnp.int32)
    ti = tail_index.astype(jnp.int32)
    ri = rel_type.astype(jnp.float32).reshape(1, B)

    xg = x.T.reshape(N * D // 16, 16)
    hidx = hi.reshape(B // _ICH, _ICH)
    tidx = ti.reshape(B // _ICH, _ICH)
    sc_gather = _make_sc_gather(B, D, N)
    hf, tf = sc_gather(xg, hidx, tidx)
    bpw = B // _NW
    ht = hf.reshape(_NW, D, bpw)
    tt = tf.reshape(_NW, D, bpw)

    relt = rel_emb.T
    w = conv_w.reshape(out_ch, 3)
    gt = fc_w.reshape(out_ch, D).T
    score, l2s = _tc_score(ht, tt, ri, relt, w, conv_b, gt, out_ch=out_ch)
    return score.reshape(B), l2s[0, 0]


# speedup vs baseline: 14.3316x; 14.3316x over previous
import functools

import jax
import jax.numpy as jnp
from jax import lax
from jax.experimental import pallas as pl
from jax.experimental.pallas import tpu as pltpu
from jax.experimental.pallas import tpu_sc as plsc

_NUM_CORES = 2
_NUM_SUBCORES = 16
_NW = _NUM_CORES * _NUM_SUBCORES


@functools.lru_cache(maxsize=None)
def _make_sc_gather(B, D):
    bpw = B // _NW
    mesh = plsc.VectorSubcoreMesh(core_axis_name="c", subcore_axis_name="s")

    @functools.partial(
        pl.kernel,
        mesh=mesh,
        out_type=[jax.ShapeDtypeStruct((B, D), jnp.float32)] * 2,
        scratch_types=[
            pltpu.VMEM((bpw,), jnp.int32),
            pltpu.VMEM((bpw,), jnp.int32),
            pltpu.VMEM((bpw // 2, D), jnp.float32),
            pltpu.VMEM((bpw // 2, D), jnp.float32),
            pltpu.SemaphoreType.DMA,
        ],
        compiler_params=pltpu.CompilerParams(needs_layout_passes=False),
    )
    def sc_gather(x3_hbm, hidx_hbm, tidx_hbm,
                  h_out, t_out,
                  hiv, tiv, hrows, trows, sem):
        wid = lax.axis_index("s") * _NUM_CORES + lax.axis_index("c")
        base = wid * bpw
        pltpu.sync_copy(hidx_hbm.at[pl.ds(base, bpw)], hiv)
        pltpu.sync_copy(tidx_hbm.at[pl.ds(base, bpw)], tiv)
        lanes = lax.broadcasted_iota(jnp.int32, (16,), 0)
        half = bpw // 2

        for p in range(2):
            def group(g, _, p=p):
                hvec = hiv[pl.ds(p * half + g * 16, 16)]
                tvec = tiv[pl.ds(p * half + g * 16, 16)]
                cps = []
                for i in range(16):
                    hidx = jnp.sum(jnp.where(lanes == i, hvec, 0))
                    tidx = jnp.sum(jnp.where(lanes == i, tvec, 0))
                    dst = pl.ds(g * 16 + i, 1)
                    cps.append(pltpu.async_copy(
                        x3_hbm.at[pl.ds(hidx >> 3, 1), hidx & 7],
                        hrows.at[dst], sem))
                    cps.append(pltpu.async_copy(
                        x3_hbm.at[pl.ds(tidx >> 3, 1), tidx & 7],
                        trows.at[dst], sem))
                for cp in cps:
                    cp.wait()
                return 0

            lax.fori_loop(0, half // 16, group, 0)
            dst = pl.ds(base + p * half, half)
            pltpu.sync_copy(hrows, h_out.at[dst])
            pltpu.sync_copy(trows, t_out.at[dst])

    return sc_gather


def _tc_score_body(h_ref, t_ref, ri_ref, rel_ref, w_ref, cb_ref, g_ref,
                   s_ref, l2_ref, *, out_ch, n_rel, l2_scale):
    i = pl.program_id(0)
    h = h_ref[...]
    t = t_ref[...]
    rows = h_ref.shape[0]
    onehot = (lax.broadcasted_iota(jnp.int32, (rows, n_rel), 1)
              == ri_ref[...]).astype(jnp.float32)
    r = jnp.dot(onehot, rel_ref[...], preferred_element_type=jnp.float32)
    acc = None
    for o in range(out_ch):
        pre = h * w_ref[o, 0] + r * w_ref[o, 1] + t * w_ref[o, 2] + cb_ref[o]
        z = jnp.maximum(pre, 0.0)
        term = z * g_ref[pl.ds(o, 1), :]
        acc = term if acc is None else acc + term
    s_ref[...] = jnp.sum(acc, axis=1, keepdims=True)
    part = (jnp.sum(h * h) + jnp.sum(t * t) + jnp.sum(r * r)) * l2_scale

    @pl.when(i == 0)
    def _():
        l2_ref[0, 0] = part

    @pl.when(i > 0)
    def _():
        l2_ref[0, 0] = l2_ref[0, 0] + part


def _tc_score(h, t, ri, rel, w, cb, g, *, out_ch, d, interpret=False):
    B = h.shape[0]
    n_rel = rel.shape[0]
    tile = 512
    nsteps = B // tile
    body = functools.partial(
        _tc_score_body, out_ch=out_ch, n_rel=n_rel,
        l2_scale=1.0 / (3.0 * B * d))
    return pl.pallas_call(
        body,
        grid=(nsteps,),
        in_specs=[
            pl.BlockSpec((tile, d), lambda i: (i, 0)),
            pl.BlockSpec((tile, d), lambda i: (i, 0)),
            pl.BlockSpec((tile, 1), lambda i: (i, 0)),
            pl.BlockSpec((n_rel, d), lambda i: (0, 0)),
            pl.BlockSpec(memory_space=pltpu.SMEM),
            pl.BlockSpec(memory_space=pltpu.SMEM),
            pl.BlockSpec((out_ch, d), lambda i: (0, 0)),
        ],
        out_specs=[
            pl.BlockSpec((tile, 1), lambda i: (i, 0)),
            pl.BlockSpec(memory_space=pltpu.SMEM),
        ],
        out_shape=[
            jax.ShapeDtypeStruct((B, 1), jnp.float32),
            jax.ShapeDtypeStruct((1, 1), jnp.float32),
        ],
        compiler_params=pltpu.CompilerParams(
            dimension_semantics=("arbitrary",)),
        interpret=interpret,
    )(h, t, ri, rel, w, cb, g)


def kernel(x, rel_emb, head_index, rel_type, tail_index, conv_w, conv_b, fc_w):
    B = head_index.shape[0]
    D = x.shape[1]
    out_ch = conv_w.shape[0]

    hi = head_index.astype(jnp.int32)
    ti = tail_index.astype(jnp.int32)
    ri = rel_type.astype(jnp.int32)

    x3 = x.reshape(x.shape[0] // 8, 8, D)
    sc_gather = _make_sc_gather(B, D)
    h, t = sc_gather(x3, hi, ti)

    w = conv_w.reshape(out_ch, 3)
    g = fc_w.reshape(out_ch, D)
    score, l2s = _tc_score(
        h, t, ri.reshape(B, 1), rel_emb, w, conv_b, g, out_ch=out_ch, d=D)
    return score.reshape(B), l2s[0, 0]


# speedup vs baseline: 15.3930x; 1.0741x over previous
import functools

import jax
import jax.numpy as jnp
from jax import lax
from jax.experimental import pallas as pl
from jax.experimental.pallas import tpu as pltpu
from jax.experimental.pallas import tpu_sc as plsc

_NUM_CORES = 2
_NUM_SUBCORES = 16
_NW = _NUM_CORES * _NUM_SUBCORES


@functools.lru_cache(maxsize=None)
def _make_sc_gather(B, D):
    bpw = B // _NW
    mesh = plsc.VectorSubcoreMesh(core_axis_name="c", subcore_axis_name="s")

    @functools.partial(
        pl.kernel,
        mesh=mesh,
        out_type=[jax.ShapeDtypeStruct((B, D), jnp.float32)] * 2,
        scratch_types=[
            pltpu.VMEM((bpw,), jnp.int32),
            pltpu.VMEM((bpw,), jnp.int32),
            pltpu.VMEM((bpw // 2, D), jnp.float32),
            pltpu.VMEM((bpw // 2, D), jnp.float32),
            pltpu.SemaphoreType.DMA,
        ],
        compiler_params=pltpu.CompilerParams(needs_layout_passes=False),
    )
    def sc_gather(x3_hbm, hidx_hbm, tidx_hbm,
                  h_out, t_out,
                  hiv, tiv, hrows, trows, sem):
        wid = lax.axis_index("s") * _NUM_CORES + lax.axis_index("c")
        base = wid * bpw
        pltpu.sync_copy(hidx_hbm.at[pl.ds(base, bpw)], hiv)
        pltpu.sync_copy(tidx_hbm.at[pl.ds(base, bpw)], tiv)
        lanes = lax.broadcasted_iota(jnp.int32, (16,), 0)
        half = bpw // 2

        for p in range(2):
            def group(g, _, p=p):
                hvec = hiv[pl.ds(p * half + g * 16, 16)]
                tvec = tiv[pl.ds(p * half + g * 16, 16)]
                cps = []
                for i in range(16):
                    hidx = jnp.sum(jnp.where(lanes == i, hvec, 0))
                    tidx = jnp.sum(jnp.where(lanes == i, tvec, 0))
                    dst = pl.ds(g * 16 + i, 1)
                    cps.append(pltpu.async_copy(
                        x3_hbm.at[pl.ds(hidx >> 3, 1), hidx & 7],
                        hrows.at[dst], sem))
                    cps.append(pltpu.async_copy(
                        x3_hbm.at[pl.ds(tidx >> 3, 1), tidx & 7],
                        trows.at[dst], sem))
                for cp in cps:
                    cp.wait()
                return 0

            lax.fori_loop(0, half // 16, group, 0)
            dst = pl.ds(base + p * half, half)
            pltpu.sync_copy(hrows, h_out.at[dst])
            pltpu.sync_copy(trows, t_out.at[dst])

    return sc_gather


def _tc_score_body(h_ref, t_ref, rie_ref, rio_ref, rel2_ref, w_ref, cb_ref,
                   g2_ref, s_ref, l2_ref, *, out_ch, n_rel, d, l2_scale):
    i = pl.program_id(0)
    h = h_ref[...]
    t = t_ref[...]
    rows = h_ref.shape[0]
    io = lax.broadcasted_iota(jnp.int32, (rows, 2 * n_rel), 1)
    oh = ((io == rie_ref[...]) | (io == rio_ref[...] + n_rel)
          ).astype(jnp.float32)
    r = jnp.dot(oh, rel2_ref[...], preferred_element_type=jnp.float32)
    acc = None
    for o in range(out_ch):
        pre = h * w_ref[o, 0] + r * w_ref[o, 1] + t * w_ref[o, 2] + cb_ref[o]
        z = jnp.maximum(pre, 0.0)
        term = z * g2_ref[pl.ds(o, 1), :]
        acc = term if acc is None else acc + term
    s0 = jnp.sum(acc[:, :d], axis=1, keepdims=True)
    s1 = jnp.sum(acc[:, d:], axis=1, keepdims=True)
    s_ref[...] = jnp.concatenate([s0, s1], axis=1)
    part = (jnp.sum(h * h) + jnp.sum(t * t) + jnp.sum(r * r)) * l2_scale

    @pl.when(i == 0)
    def _():
        l2_ref[0, 0] = part

    @pl.when(i > 0)
    def _():
        l2_ref[0, 0] = l2_ref[0, 0] + part


def _tc_score(h2, t2, rie, rio, rel2, w, cb, g2, *, out_ch, d,
              interpret=False):
    B2 = h2.shape[0]
    n_rel = rel2.shape[0] // 2
    tile = 512
    nsteps = B2 // tile
    body = functools.partial(
        _tc_score_body, out_ch=out_ch, n_rel=n_rel, d=d,
        l2_scale=1.0 / (3.0 * B2 * 2 * d))
    return pl.pallas_call(
        body,
        grid=(nsteps,),
        in_specs=[
            pl.BlockSpec((tile, 2 * d), lambda i: (i, 0)),
            pl.BlockSpec((tile, 2 * d), lambda i: (i, 0)),
            pl.BlockSpec((tile, 1), lambda i: (i, 0)),
            pl.BlockSpec((tile, 1), lambda i: (i, 0)),
            pl.BlockSpec((2 * n_rel, 2 * d), lambda i: (0, 0)),
            pl.BlockSpec(memory_space=pltpu.SMEM),
            pl.BlockSpec(memory_space=pltpu.SMEM),
            pl.BlockSpec((out_ch, 2 * d), lambda i: (0, 0)),
        ],
        out_specs=[
            pl.BlockSpec((tile, 2), lambda i: (i, 0)),
            pl.BlockSpec(memory_space=pltpu.SMEM),
        ],
        out_shape=[
            jax.ShapeDtypeStruct((B2, 2), jnp.float32),
            jax.ShapeDtypeStruct((1, 1), jnp.float32),
        ],
        compiler_params=pltpu.CompilerParams(
            dimension_semantics=("arbitrary",)),
        interpret=interpret,
    )(h2, t2, rie, rio, rel2, w, cb, g2)


def kernel(x, rel_emb, head_index, rel_type, tail_index, conv_w, conv_b, fc_w):
    B = head_index.shape[0]
    D = x.shape[1]
    out_ch = conv_w.shape[0]

    hi = head_index.astype(jnp.int32)
    ti = tail_index.astype(jnp.int32)
    ri = rel_type.astype(jnp.int32)

    x3 = x.reshape(x.shape[0] // 8, 8, D)
    sc_gather = _make_sc_gather(B, D)
    h, t = sc_gather(x3, hi, ti)

    h2 = h.reshape(B // 2, 2 * D)
    t2 = t.reshape(B // 2, 2 * D)
    rie = ri.reshape(B // 2, 2)[:, 0:1]
    rio = ri.reshape(B // 2, 2)[:, 1:2]
    zr = jnp.zeros_like(rel_emb)
    rel2 = jnp.concatenate([
        jnp.concatenate([rel_emb, zr], axis=1),
        jnp.concatenate([zr, rel_emb], axis=1)], axis=0)
    w = conv_w.reshape(out_ch, 3)
    g = fc_w.reshape(out_ch, D)
    g2 = jnp.concatenate([g, g], axis=1)
    score2, l2s = _tc_score(
        h2, t2, rie, rio, rel2, w, conv_b, g2, out_ch=out_ch, d=D)
    return score2.reshape(B), l2s[0, 0]


# speedup vs baseline: 15.4839x; 1.0059x over previous
import functools

import jax
import jax.numpy as jnp
from jax import lax
from jax.experimental import pallas as pl
from jax.experimental.pallas import tpu as pltpu
from jax.experimental.pallas import tpu_sc as plsc

_NUM_CORES = 2
_NUM_SUBCORES = 16
_NW = _NUM_CORES * _NUM_SUBCORES


@functools.lru_cache(maxsize=None)
def _make_sc_gather(B, D):
    bpw = B // _NW
    mesh = plsc.VectorSubcoreMesh(core_axis_name="c", subcore_axis_name="s")

    @functools.partial(
        pl.kernel,
        mesh=mesh,
        out_type=[jax.ShapeDtypeStruct((B, D), jnp.float32)] * 2,
        scratch_types=[
            pltpu.VMEM((bpw,), jnp.int32),
            pltpu.VMEM((bpw,), jnp.int32),
            pltpu.VMEM((bpw // 2, D), jnp.float32),
            pltpu.VMEM((bpw // 2, D), jnp.float32),
            pltpu.SemaphoreType.DMA,
        ],
        compiler_params=pltpu.CompilerParams(needs_layout_passes=False),
    )
    def sc_gather(x3_hbm, hidx_hbm, tidx_hbm,
                  h_out, t_out,
                  hiv, tiv, hrows, trows, sem):
        wid = lax.axis_index("s") * _NUM_CORES + lax.axis_index("c")
        base = wid * bpw
        pltpu.sync_copy(hidx_hbm.at[pl.ds(base, bpw)], hiv)
        pltpu.sync_copy(tidx_hbm.at[pl.ds(base, bpw)], tiv)
        lanes = lax.broadcasted_iota(jnp.int32, (16,), 0)
        half = bpw // 2

        for p in range(2):
            def group(g, _, p=p):
                hvec = hiv[pl.ds(p * half + g * 16, 16)]
                tvec = tiv[pl.ds(p * half + g * 16, 16)]
                cps = []
                for i in range(16):
                    hidx = jnp.sum(jnp.where(lanes == i, hvec, 0))
                    tidx = jnp.sum(jnp.where(lanes == i, tvec, 0))
                    dst = pl.ds(g * 16 + i, 1)
                    cps.append(pltpu.async_copy(
                        x3_hbm.at[pl.ds(hidx >> 3, 1), hidx & 7],
                        hrows.at[dst], sem))
                    cps.append(pltpu.async_copy(
                        x3_hbm.at[pl.ds(tidx >> 3, 1), tidx & 7],
                        trows.at[dst], sem))
                for cp in cps:
                    cp.wait()
                return 0

            lax.fori_loop(0, half // 16, group, 0)
            dst = pl.ds(base + p * half, half)
            pltpu.sync_copy(hrows, h_out.at[dst])
            pltpu.sync_copy(trows, t_out.at[dst])

    return sc_gather


def _tc_score_body(h_ref, t_ref, rie_ref, rio_ref, rel2_ref, w_ref, cb_ref,
                   g2_ref, s_ref, l2_ref, *, out_ch, n_rel, d, l2_scale):
    i = pl.program_id(0)
    h = h_ref[...]
    t = t_ref[...]
    rows = h_ref.shape[0]
    io = lax.broadcasted_iota(jnp.int32, (rows, 2 * n_rel), 1)
    oh = ((io == rie_ref[...]) | (io == rio_ref[...] + n_rel)
          ).astype(jnp.float32)
    r = jnp.dot(oh, rel2_ref[...], preferred_element_type=jnp.float32)
    acc = None
    for o in range(out_ch):
        pre = h * w_ref[o, 0] + r * w_ref[o, 1] + t * w_ref[o, 2] + cb_ref[o]
        z = jnp.maximum(pre, 0.0)
        term = z * g2_ref[pl.ds(o, 1), :]
        acc = term if acc is None else acc + term
    s0 = jnp.sum(acc[:, :d], axis=1, keepdims=True)
    s1 = jnp.sum(acc[:, d:], axis=1, keepdims=True)
    s_ref[...] = jnp.concatenate([s0, s1], axis=1)
    part = (jnp.sum(h * h) + jnp.sum(t * t) + jnp.sum(r * r)) * l2_scale

    @pl.when(i == 0)
    def _():
        l2_ref[0, 0] = part

    @pl.when(i > 0)
    def _():
        l2_ref[0, 0] = l2_ref[0, 0] + part


def _tc_score(h2, t2, rie, rio, rel2, w, cb, g2, *, out_ch, d,
              interpret=False):
    B2 = h2.shape[0]
    n_rel = rel2.shape[0] // 2
    tile = 1024
    nsteps = B2 // tile
    body = functools.partial(
        _tc_score_body, out_ch=out_ch, n_rel=n_rel, d=d,
        l2_scale=1.0 / (3.0 * B2 * 2 * d))
    return pl.pallas_call(
        body,
        grid=(nsteps,),
        in_specs=[
            pl.BlockSpec((tile, 2 * d), lambda i: (i, 0)),
            pl.BlockSpec((tile, 2 * d), lambda i: (i, 0)),
            pl.BlockSpec((tile, 1), lambda i: (i, 0)),
            pl.BlockSpec((tile, 1), lambda i: (i, 0)),
            pl.BlockSpec((2 * n_rel, 2 * d), lambda i: (0, 0)),
            pl.BlockSpec(memory_space=pltpu.SMEM),
            pl.BlockSpec(memory_space=pltpu.SMEM),
            pl.BlockSpec((out_ch, 2 * d), lambda i: (0, 0)),
        ],
        out_specs=[
            pl.BlockSpec((tile, 2), lambda i: (i, 0)),
            pl.BlockSpec(memory_space=pltpu.SMEM),
        ],
        out_shape=[
            jax.ShapeDtypeStruct((B2, 2), jnp.float32),
            jax.ShapeDtypeStruct((1, 1), jnp.float32),
        ],
        compiler_params=pltpu.CompilerParams(
            dimension_semantics=("arbitrary",)),
        interpret=interpret,
    )(h2, t2, rie, rio, rel2, w, cb, g2)


def kernel(x, rel_emb, head_index, rel_type, tail_index, conv_w, conv_b, fc_w):
    B = head_index.shape[0]
    D = x.shape[1]
    out_ch = conv_w.shape[0]

    hi = head_index.astype(jnp.int32)
    ti = tail_index.astype(jnp.int32)
    ri = rel_type.astype(jnp.int32)

    x3 = x.reshape(x.shape[0] // 8, 8, D)
    sc_gather = _make_sc_gather(B, D)
    h, t = sc_gather(x3, hi, ti)

    h2 = h.reshape(B // 2, 2 * D)
    t2 = t.reshape(B // 2, 2 * D)
    rie = ri.reshape(B // 2, 2)[:, 0:1]
    rio = ri.reshape(B // 2, 2)[:, 1:2]
    zr = jnp.zeros_like(rel_emb)
    rel2 = jnp.concatenate([
        jnp.concatenate([rel_emb, zr], axis=1),
        jnp.concatenate([zr, rel_emb], axis=1)], axis=0)
    w = conv_w.reshape(out_ch, 3)
    g = fc_w.reshape(out_ch, D)
    g2 = jnp.concatenate([g, g], axis=1)
    score2, l2s = _tc_score(
        h2, t2, rie, rio, rel2, w, conv_b, g2, out_ch=out_ch, d=D)
    return score2.reshape(B), l2s[0, 0]
